# baseline (speedup 1.0000x reference)
import functools

import jax
import jax.numpy as jnp
from jax import lax
from jax.experimental import pallas as pl
from jax.experimental.pallas import tpu as pltpu
from jax.experimental.pallas import tpu_sc as plsc

N_NODES = 10000
N_EDGES = 320000
D = 128

NC = 2
NS = 16
NW = NC * NS
WIN = 120
WPW = 84
E_PAD = NW * WPW * WIN
NBUF = 3
IBUF = 6
UNROLL = 6

N_PAD = 10240
ROWS_PER_TILE = N_PAD // NS
DUMP = N_PAD - N_NODES


def _make_sc_agg(with_cnt):
    mesh = plsc.VectorSubcoreMesh(core_axis_name="c", subcore_axis_name="s")

    out_type = [jax.ShapeDtypeStruct((NC, N_PAD, D), jnp.float32)]
    if with_cnt:
        out_type.append(jax.ShapeDtypeStruct((NC, N_PAD), jnp.float32))

    scratch = [pltpu.VMEM_SHARED((N_PAD, D), jnp.float32)]
    scratch += [pltpu.VMEM((2, WIN), jnp.int32) for _ in range(IBUF)]
    scratch += [pltpu.VMEM((WIN, D), jnp.float32) for _ in range(NBUF)]
    scratch += [pltpu.SemaphoreType.DMA for _ in range(IBUF + 2 * NBUF)]
    if with_cnt:
        scratch += [
            pltpu.VMEM_SHARED((N_PAD,), jnp.float32),
            pltpu.VMEM((WIN,), jnp.float32),
        ]

    def body(*refs):
        it = iter(refs)
        x_hbm = next(it); sd_hbm = next(it)
        zrows_hbm = next(it)
        zcnt_hbm = next(it) if with_cnt else None
        agg_out = next(it)
        cnt_out = next(it) if with_cnt else None
        agg_sh = next(it)
        idxb = [next(it) for _ in range(IBUF)]
        rows = [next(it) for _ in range(NBUF)]
        isem = [next(it) for _ in range(IBUF)]
        rsem = [next(it) for _ in range(NBUF)]
        ssem = [next(it) for _ in range(NBUF)]
        if with_cnt:
            cnt_sh = next(it); ones_v = next(it)

        cid = lax.axis_index("c")
        sid = lax.axis_index("s")
        wid = cid * NS + sid
        r0 = sid * ROWS_PER_TILE
        base = wid * WPW

        pltpu.sync_copy(zrows_hbm, agg_sh.at[pl.ds(r0, ROWS_PER_TILE), :])
        if with_cnt:
            pltpu.sync_copy(zcnt_hbm, cnt_sh.at[pl.ds(r0, ROWS_PER_TILE)])
            for j in range(7):
                ones_v[pl.ds(16 * j, 16)] = jnp.full((16,), 1.0, jnp.float32)
            ones_v[pl.ds(WIN - 16, 16)] = jnp.full((16,), 1.0, jnp.float32)

        for i in range(IBUF):
            pltpu.async_copy(sd_hbm.at[base + i], idxb[i], isem[i])
        for b in range(2):
            pltpu.make_async_copy(sd_hbm.at[0], idxb[b], isem[b]).wait()
            pltpu.async_copy(x_hbm.at[idxb[b].at[0]], rows[b], rsem[b])
        plsc.subcore_barrier()

        def group(g, carry):
            for k in range(UNROLL):
                w = g * UNROLL + k
                rb = k % NBUF
                rb1 = (k - 1) % NBUF
                s_idx = k % IBUF
                s_ref = (k + 5) % IBUF
                s_g = (k + 2) % IBUF

                pltpu.make_async_copy(
                    x_hbm.at[pl.ds(0, WIN), :], rows[rb], rsem[rb]).wait()

                if with_cnt:
                    pltpu.sync_copy(ones_v, cnt_sh.at[idxb[s_idx].at[1]],
                                    add=True)
                pltpu.async_copy(rows[rb], agg_sh.at[idxb[s_idx].at[1]],
                                 ssem[rb], add=True)

                @pl.when(w >= 1)
                def _():
                    pltpu.make_async_copy(
                        rows[rb1], agg_sh.at[idxb[s_idx].at[1]],
                        ssem[rb1]).wait()

                @pl.when((w >= 1) & (w + 5 < WPW))
                def _():
                    pltpu.async_copy(sd_hbm.at[base + w + 5],
                                     idxb[s_ref], isem[s_ref])

                @pl.when(w + 2 < WPW)
                def _():
                    pltpu.make_async_copy(
                        sd_hbm.at[0], idxb[s_g], isem[s_g]).wait()
                    pltpu.async_copy(x_hbm.at[idxb[s_g].at[0]],
                                     rows[(k + 2) % NBUF], rsem[(k + 2) % NBUF])
            return carry

        lax.fori_loop(0, WPW // UNROLL, group, 0)

        pltpu.make_async_copy(
            rows[(WPW - 1) % NBUF],
            agg_sh.at[idxb[(WPW - 1) % IBUF].at[1]],
            ssem[(WPW - 1) % NBUF]).wait()
        plsc.subcore_barrier()

        pltpu.sync_copy(agg_sh.at[pl.ds(r0, ROWS_PER_TILE), :],
                        agg_out.at[cid, pl.ds(r0, ROWS_PER_TILE), :])
        if with_cnt:
            pltpu.sync_copy(cnt_sh.at[pl.ds(r0, ROWS_PER_TILE)],
                            cnt_out.at[cid, pl.ds(r0, ROWS_PER_TILE)])

    return functools.partial(pl.kernel, mesh=mesh,
                             out_type=out_type,
                             scratch_types=scratch)(body)


_sc_agg_cnt = _make_sc_agg(with_cnt=True)
_sc_agg = _make_sc_agg(with_cnt=False)


_R = 1024


def _dense_self(x, wrT, b):
    def body(x_ref, wr_ref, b_ref, o_ref):
        o_ref[...] = (jnp.dot(x_ref[...], wr_ref[...],
                              preferred_element_type=jnp.float32)
                      + b_ref[...])

    return pl.pallas_call(
        body,
        grid=(N_PAD // _R,),
        in_specs=[
            pl.BlockSpec((_R, D), lambda i: (i, 0)),
            pl.BlockSpec((D, D), lambda i: (0, 0)),
            pl.BlockSpec((1, D), lambda i: (0, 0)),
        ],
        out_specs=pl.BlockSpec((_R, D), lambda i: (i, 0)),
        out_shape=jax.ShapeDtypeStruct((N_NODES, D), jnp.float32),
    )(x, wrT, b)


def _dense_agg(aggp, cntp, slf, wlT):
    def body(aggp_ref, cntp_ref, s_ref, wl_ref, o_ref):
        agg = aggp_ref[0] + aggp_ref[1]
        cnt = cntp_ref[0] + cntp_ref[1]
        inv = 1.0 / jnp.maximum(cnt, 1.0)
        mean = agg * inv[:, None]
        acc = jnp.dot(mean, wl_ref[...], preferred_element_type=jnp.float32)
        o_ref[...] = jnp.maximum(acc + s_ref[...], 0.0)

    return pl.pallas_call(
        body,
        grid=(N_PAD // _R,),
        in_specs=[
            pl.BlockSpec((NC, _R, D), lambda i: (0, i, 0)),
            pl.BlockSpec((NC, _R), lambda i: (0, i)),
            pl.BlockSpec((_R, D), lambda i: (i, 0)),
            pl.BlockSpec((D, D), lambda i: (0, 0)),
        ],
        out_specs=pl.BlockSpec((_R, D), lambda i: (i, 0)),
        out_shape=jax.ShapeDtypeStruct((N_NODES, D), jnp.float32),
    )(aggp, cntp, slf, wlT)


def kernel(x, edge_index, W1_l, W1_r, b1, W2_l, W2_r, b2):
    src = edge_index[0].astype(jnp.int32)
    dst = edge_index[1].astype(jnp.int32)
    npad_e = E_PAD - N_EDGES
    pad_src = (jnp.arange(npad_e, dtype=jnp.int32) * 4001) % N_NODES
    pad_dst = jnp.arange(npad_e, dtype=jnp.int32) % DUMP + N_NODES
    srcp = jnp.concatenate([src, pad_src]).reshape(NW * WPW, WIN)
    dstp = jnp.concatenate([dst, pad_dst]).reshape(NW * WPW, WIN)
    sd = jnp.stack([srcp, dstp], axis=1)
    zrows = jnp.zeros((ROWS_PER_TILE, D), jnp.float32)
    zcnt = jnp.zeros((ROWS_PER_TILE,), jnp.float32)

    aggp1, cntp = _sc_agg_cnt(x, sd, zrows, zcnt)
    self1 = _dense_self(x, W1_r.T, b1.reshape(1, D))
    h = _dense_agg(aggp1, cntp, self1, W1_l.T)
    aggp2 = _sc_agg(h, sd, zrows)
    if isinstance(aggp2, (list, tuple)):
        aggp2 = aggp2[0]
    self2 = _dense_self(h, W2_r.T, b2.reshape(1, D))
    return _dense_agg(aggp2, cntp, self2, W2_l.T)


# speedup vs baseline: 1.0470x; 1.0470x over previous
import functools

import jax
import jax.numpy as jnp
from jax import lax
from jax.experimental import pallas as pl
from jax.experimental.pallas import tpu as pltpu
from jax.experimental.pallas import tpu_sc as plsc

N_NODES = 10000
N_EDGES = 320000
D = 128

NC = 2
NS = 16
NW = NC * NS
WIN = 120
WPW = 84
E_PAD = NW * WPW * WIN
NBUF = 3
IBUF = 4
UNROLL = 12

N_PAD = 10240
ROWS_PER_TILE = N_PAD // NS
DUMP = N_PAD - N_NODES


def _make_sc_agg(with_cnt):
    mesh = plsc.VectorSubcoreMesh(core_axis_name="c", subcore_axis_name="s")

    out_type = [jax.ShapeDtypeStruct((NC, N_PAD, D), jnp.float32)]
    if with_cnt:
        out_type.append(jax.ShapeDtypeStruct((NC, N_PAD), jnp.float32))

    scratch = [pltpu.VMEM_SHARED((N_PAD, D), jnp.float32)]
    scratch += [pltpu.VMEM((2, WIN), jnp.int32) for _ in range(IBUF)]
    scratch += [pltpu.VMEM((WIN, D), jnp.float32) for _ in range(NBUF)]
    scratch += [pltpu.SemaphoreType.DMA for _ in range(IBUF + NBUF)]
    if with_cnt:
        scratch += [
            pltpu.VMEM_SHARED((N_PAD,), jnp.float32),
            pltpu.VMEM((WIN,), jnp.float32),
        ]

    def body(*refs):
        it = iter(refs)
        x_hbm = next(it); sd_hbm = next(it)
        zrows_hbm = next(it)
        zcnt_hbm = next(it) if with_cnt else None
        agg_out = next(it)
        cnt_out = next(it) if with_cnt else None
        agg_sh = next(it)
        idxb = [next(it) for _ in range(IBUF)]
        rows = [next(it) for _ in range(NBUF)]
        isem = [next(it) for _ in range(IBUF)]
        rsem = [next(it) for _ in range(NBUF)]
        if with_cnt:
            cnt_sh = next(it); ones_v = next(it)

        cid = lax.axis_index("c")
        sid = lax.axis_index("s")
        wid = cid * NS + sid
        r0 = sid * ROWS_PER_TILE
        base = wid * WPW

        pltpu.sync_copy(zrows_hbm, agg_sh.at[pl.ds(r0, ROWS_PER_TILE), :])
        if with_cnt:
            pltpu.sync_copy(zcnt_hbm, cnt_sh.at[pl.ds(r0, ROWS_PER_TILE)])
            for j in range(WIN // 8 // 2):
                ones_v[pl.ds(16 * j, 16)] = jnp.full((16,), 1.0, jnp.float32)
            ones_v[pl.ds(WIN - 16, 16)] = jnp.full((16,), 1.0, jnp.float32)

        for i in range(IBUF):
            pltpu.async_copy(sd_hbm.at[base + i], idxb[i], isem[i])
        for b in range(NBUF):
            pltpu.make_async_copy(sd_hbm.at[0], idxb[b], isem[b]).wait()
            pltpu.async_copy(x_hbm.at[idxb[b].at[0]], rows[b], rsem[b])
        plsc.subcore_barrier()

        def group(g, carry):
            for k in range(UNROLL):
                w = g * UNROLL + k
                rb = k % NBUF
                ib = k % IBUF
                ib3 = (k + NBUF) % IBUF
                pltpu.make_async_copy(
                    x_hbm.at[pl.ds(0, WIN), :], rows[rb], rsem[rb]).wait()
                pltpu.sync_copy(rows[rb], agg_sh.at[idxb[ib].at[1]], add=True)
                if with_cnt:
                    pltpu.sync_copy(ones_v, cnt_sh.at[idxb[ib].at[1]],
                                    add=True)

                @pl.when(w + IBUF < WPW)
                def _():
                    pltpu.async_copy(sd_hbm.at[base + w + IBUF],
                                     idxb[ib], isem[ib])

                @pl.when(w + NBUF < WPW)
                def _():
                    pltpu.make_async_copy(
                        sd_hbm.at[0], idxb[ib3], isem[ib3]).wait()
                    pltpu.async_copy(x_hbm.at[idxb[ib3].at[0]],
                                     rows[rb], rsem[rb])
            return carry

        lax.fori_loop(0, WPW // UNROLL, group, 0)
        plsc.subcore_barrier()

        pltpu.sync_copy(agg_sh.at[pl.ds(r0, ROWS_PER_TILE), :],
                        agg_out.at[cid, pl.ds(r0, ROWS_PER_TILE), :])
        if with_cnt:
            pltpu.sync_copy(cnt_sh.at[pl.ds(r0, ROWS_PER_TILE)],
                            cnt_out.at[cid, pl.ds(r0, ROWS_PER_TILE)])

    return functools.partial(pl.kernel, mesh=mesh,
                             out_type=out_type,
                             scratch_types=scratch)(body)


_sc_agg_cnt = _make_sc_agg(with_cnt=True)
_sc_agg = _make_sc_agg(with_cnt=False)


_R = 1024


def _dense_self(x, wrT, b):
    def body(x_ref, wr_ref, b_ref, o_ref):
        o_ref[...] = (jnp.dot(x_ref[...], wr_ref[...],
                              preferred_element_type=jnp.float32)
                      + b_ref[...])

    return pl.pallas_call(
        body,
        grid=(N_PAD // _R,),
        in_specs=[
            pl.BlockSpec((_R, D), lambda i: (i, 0)),
            pl.BlockSpec((D, D), lambda i: (0, 0)),
            pl.BlockSpec((1, D), lambda i: (0, 0)),
        ],
        out_specs=pl.BlockSpec((_R, D), lambda i: (i, 0)),
        out_shape=jax.ShapeDtypeStruct((N_NODES, D), jnp.float32),
    )(x, wrT, b)


def _dense_agg(aggp, cntp, slf, wlT):
    def body(aggp_ref, cntp_ref, s_ref, wl_ref, o_ref):
        agg = aggp_ref[0] + aggp_ref[1]
        cnt = cntp_ref[0] + cntp_ref[1]
        inv = 1.0 / jnp.maximum(cnt, 1.0)
        mean = agg * inv[:, None]
        acc = jnp.dot(mean, wl_ref[...], preferred_element_type=jnp.float32)
        o_ref[...] = jnp.maximum(acc + s_ref[...], 0.0)

    return pl.pallas_call(
        body,
        grid=(N_PAD // _R,),
        in_specs=[
            pl.BlockSpec((NC, _R, D), lambda i: (0, i, 0)),
            pl.BlockSpec((NC, _R), lambda i: (0, i)),
            pl.BlockSpec((_R, D), lambda i: (i, 0)),
            pl.BlockSpec((D, D), lambda i: (0, 0)),
        ],
        out_specs=pl.BlockSpec((_R, D), lambda i: (i, 0)),
        out_shape=jax.ShapeDtypeStruct((N_NODES, D), jnp.float32),
    )(aggp, cntp, slf, wlT)


def kernel(x, edge_index, W1_l, W1_r, b1, W2_l, W2_r, b2):
    src = edge_index[0].astype(jnp.int32)
    dst = edge_index[1].astype(jnp.int32)
    npad_e = E_PAD - N_EDGES
    pad_src = (jnp.arange(npad_e, dtype=jnp.int32) * 4001) % N_NODES
    pad_dst = jnp.arange(npad_e, dtype=jnp.int32) % DUMP + N_NODES
    srcp = jnp.concatenate([src, pad_src]).reshape(NW * WPW, WIN)
    dstp = jnp.concatenate([dst, pad_dst]).reshape(NW * WPW, WIN)
    sd = jnp.stack([srcp, dstp], axis=1)
    zrows = jnp.zeros((ROWS_PER_TILE, D), jnp.float32)
    zcnt = jnp.zeros((ROWS_PER_TILE,), jnp.float32)

    aggp1, cntp = _sc_agg_cnt(x, sd, zrows, zcnt)
    self1 = _dense_self(x, W1_r.T, b1.reshape(1, D))
    h = _dense_agg(aggp1, cntp, self1, W1_l.T)
    aggp2 = _sc_agg(h, sd, zrows)
    if isinstance(aggp2, (list, tuple)):
        aggp2 = aggp2[0]
    self2 = _dense_self(h, W2_r.T, b2.reshape(1, D))
    return _dense_agg(aggp2, cntp, self2, W2_l.T)
